# baseline (speedup 1.0000x reference)
import functools

import jax
import jax.numpy as jnp
from jax import lax
from jax.experimental import pallas as pl
from jax.experimental.pallas import tpu as pltpu
from jax.experimental.pallas import tpu_sc as plsc

B = 1024
L = 200
D = 128
NC = 2
NS = 16
NW = NC * NS
ROWS_PER_W = (B * L) // NW
SUB = 128
NSUB = ROWS_PER_W // SUB
NPAIR = NSUB // 2
NBUF = 3
NCOMBO = 2 * L
NG = D // 16
CB = 40
NCB = NCOMBO // CB


def _make_kernel():
  mesh = plsc.VectorSubcoreMesh(
      core_axis_name="c", subcore_axis_name="s", num_cores=NC, num_subcores=NS
  )

  @functools.partial(
      pl.kernel,
      out_type=jax.ShapeDtypeStruct((NW, NSUB, SUB, D), jnp.float32),
      mesh=mesh,
      scratch_types=[
          pltpu.VMEM((NSUB, SUB), jnp.int32),
          pltpu.VMEM((NSUB, SUB), jnp.int32),
          pltpu.VMEM((CB, D), jnp.float32),
          pltpu.VMEM((2, D), jnp.float32),
          pltpu.VMEM_SHARED((NCOMBO, D), jnp.float32),
          [pltpu.VMEM((2, SUB, D), jnp.float32) for _ in range(NBUF)],
          [pltpu.SemaphoreType.DMA for _ in range(NBUF)],
      ],
  )
  def bert_embed(ids_hbm, sids_hbm, tok_hbm, pos_hbm, seg_hbm,
                 out_hbm, idx_v, cidx_v, cb_v, seg2_v, combo_sh, bufs, sems):
    cid = lax.axis_index("c")
    sid = lax.axis_index("s")
    wid = sid * NC + cid

    @pl.when(sid < NCB)
    def _build():
      lstart = (sid % (NCB // 2)) * CB
      seg_row = sid // (NCB // 2)
      pltpu.sync_copy(pos_hbm.at[pl.ds(lstart, CB)], cb_v)
      pltpu.sync_copy(seg_hbm, seg2_v)
      sg = [seg2_v[seg_row, pl.ds(g * 16, 16)] for g in range(NG)]

      def add_seg(r, carry):
        for g in range(NG):
          plsc.addupdate(cb_v.at[r, pl.ds(g * 16, 16)], sg[g])
        return carry

      lax.fori_loop(0, CB, add_seg, 0)
      pltpu.sync_copy(cb_v, combo_sh.at[pl.ds(sid * CB, CB)])

    pltpu.sync_copy(ids_hbm.at[wid], idx_v)
    pltpu.sync_copy(sids_hbm.at[wid], cidx_v)
    plsc.subcore_barrier()

    lanes = lax.iota(jnp.int32, 16)

    def mk_cidx(a):
      for g in range(SUB // 16):
        lbase = (a * SUB + g * 16) % L
        lv = jnp.full((16,), lbase, jnp.int32) + lanes
        lv = jnp.where(lv >= L, lv - L, lv)
        sl = (g * 16, 16)
        cidx_v[a, pl.ds(*sl)] = cidx_v[a, pl.ds(*sl)] * L + lv

    c = [None] * NPAIR
    t = [None] * NPAIR
    w = [None] * NPAIR
    for j in range(NPAIR + 2):
      a = j
      if a < NPAIR:
        ba = a % NBUF
        if a >= NBUF:
          w[a - NBUF].wait()
        mk_cidx(2 * a)
        mk_cidx(2 * a + 1)
        c[a] = [
            pltpu.async_copy(
                combo_sh.at[cidx_v.at[2 * a + h]], bufs[ba].at[h], sems[ba])
            for h in range(2)
        ]
      b = j - 1
      if 0 <= b < NPAIR:
        bb = b % NBUF
        c[b][0].wait()
        c[b][1].wait()
        t[b] = [
            pltpu.async_copy(
                tok_hbm.at[idx_v.at[2 * b + h]], bufs[bb].at[h], sems[bb],
                add=True)
            for h in range(2)
        ]
      d = j - 2
      if 0 <= d < NPAIR:
        bd = d % NBUF
        t[d][0].wait()
        t[d][1].wait()
        w[d] = pltpu.async_copy(
            bufs[bd], out_hbm.at[wid, pl.ds(2 * d, 2)], sems[bd])
    for d in range(NPAIR - NBUF, NPAIR):
      w[d].wait()

  return bert_embed


_bert_embed = _make_kernel()


@jax.jit
def kernel(input_ids, segment_ids, token_table, position_table, segment_table):
  ids = input_ids.astype(jnp.int32).reshape(NW, NSUB, SUB)
  sids = segment_ids.astype(jnp.int32).reshape(NW, NSUB, SUB)
  out = _bert_embed(ids, sids, token_table, position_table, segment_table)
  return out.reshape(B, L, D)


# speedup vs baseline: 1.0076x; 1.0076x over previous
import functools

import jax
import jax.numpy as jnp
from jax import lax
from jax.experimental import pallas as pl
from jax.experimental.pallas import tpu as pltpu
from jax.experimental.pallas import tpu_sc as plsc

B = 1024
L = 200
D = 128
NC = 2
NS = 16
NW = NC * NS
ROWS_PER_W = (B * L) // NW
SUB = 128
NSUB = ROWS_PER_W // SUB
NBUF = 6
NCOMBO = 2 * L
NG = D // 16
CB = 40
NCB = NCOMBO // CB


def _make_kernel():
  mesh = plsc.VectorSubcoreMesh(
      core_axis_name="c", subcore_axis_name="s", num_cores=NC, num_subcores=NS
  )

  @functools.partial(
      pl.kernel,
      out_type=jax.ShapeDtypeStruct((NW, NSUB, SUB, D), jnp.float32),
      mesh=mesh,
      scratch_types=[
          pltpu.VMEM((NSUB, SUB), jnp.int32),
          pltpu.VMEM((NSUB, SUB), jnp.int32),
          pltpu.VMEM((CB, D), jnp.float32),
          pltpu.VMEM((2, D), jnp.float32),
          pltpu.VMEM_SHARED((NCOMBO, D), jnp.float32),
          [pltpu.VMEM((SUB, D), jnp.float32) for _ in range(NBUF)],
          [pltpu.SemaphoreType.DMA for _ in range(NBUF)],
      ],
  )
  def bert_embed(ids_hbm, sids_hbm, tok_hbm, pos_hbm, seg_hbm,
                 out_hbm, idx_v, cidx_v, cb_v, seg2_v, combo_sh, bufs, sems):
    cid = lax.axis_index("c")
    sid = lax.axis_index("s")
    wid = sid * NC + cid

    @pl.when(sid < NCB)
    def _build():
      lstart = (sid % (NCB // 2)) * CB
      seg_row = sid // (NCB // 2)
      pltpu.sync_copy(pos_hbm.at[pl.ds(lstart, CB)], cb_v)
      pltpu.sync_copy(seg_hbm, seg2_v)
      sg = [seg2_v[seg_row, pl.ds(g * 16, 16)] for g in range(NG)]

      def add_seg(r, carry):
        for g in range(NG):
          plsc.addupdate(cb_v.at[r, pl.ds(g * 16, 16)], sg[g])
        return carry

      lax.fori_loop(0, CB, add_seg, 0)
      pltpu.sync_copy(cb_v, combo_sh.at[pl.ds(sid * CB, CB)])

    pltpu.sync_copy(ids_hbm.at[wid], idx_v)
    pltpu.sync_copy(sids_hbm.at[wid], cidx_v)
    plsc.subcore_barrier()

    lanes = lax.iota(jnp.int32, 16)

    def mk_cidx(a):
      for g in range(SUB // 16):
        lbase = (a * SUB + g * 16) % L
        lv = jnp.full((16,), lbase, jnp.int32) + lanes
        lv = jnp.where(lv >= L, lv - L, lv)
        sl = (g * 16, 16)
        cidx_v[a, pl.ds(*sl)] = cidx_v[a, pl.ds(*sl)] * L + lv

    c = [None] * NSUB
    t = [None] * NSUB
    w = [None] * NSUB
    for j in range(NSUB + 2):
      a = j
      if a < NSUB:
        ba = a % NBUF
        if a >= NBUF:
          w[a - NBUF].wait()
        mk_cidx(a)
        c[a] = pltpu.async_copy(
            combo_sh.at[cidx_v.at[a]], bufs[ba], sems[ba])
      b = j - 1
      if 0 <= b < NSUB:
        bb = b % NBUF
        c[b].wait()
        t[b] = pltpu.async_copy(
            tok_hbm.at[idx_v.at[b]], bufs[bb], sems[bb], add=True)
      d = j - 2
      if 0 <= d < NSUB:
        bd = d % NBUF
        t[d].wait()
        w[d] = pltpu.async_copy(bufs[bd], out_hbm.at[wid, d], sems[bd])
    for d in range(NSUB - NBUF, NSUB):
      w[d].wait()

  return bert_embed


_bert_embed = _make_kernel()


@jax.jit
def kernel(input_ids, segment_ids, token_table, position_table, segment_table):
  ids = input_ids.astype(jnp.int32).reshape(NW, NSUB, SUB)
  sids = segment_ids.astype(jnp.int32).reshape(NW, NSUB, SUB)
  out = _bert_embed(ids, sids, token_table, position_table, segment_table)
  return out.reshape(B, L, D)
